# baseline (speedup 1.0000x reference)
import functools

import jax
import jax.numpy as jnp
from jax import lax
from jax.experimental import pallas as pl
from jax.experimental.pallas import tpu as pltpu
from jax.experimental.pallas import tpu_sc as plsc

N = 10000
NP = 10240
E = 320000
D_IN = 128
D_HID = 128
D_OUT = 64
EPS = 1e-5

NC = 2
NS = 16
NW = NC * NS
EC = 64
EP = 327680
RW = EP // (NW * EC)
GR = 32
RPT = NP // NS


def _seg_sum_kernel(d, with_cnt):
    mesh = plsc.VectorSubcoreMesh(
        core_axis_name="c", subcore_axis_name="s", num_cores=NC, num_subcores=NS)

    out_type = [jax.ShapeDtypeStruct((NC, NP, d), jnp.float32)]
    scratch = [
        pltpu.VMEM_SHARED((NP, d), jnp.float32),
        pltpu.VMEM((GR, EC), jnp.int32),
        pltpu.VMEM((GR, EC), jnp.int32),
        pltpu.VMEM((EC, d), jnp.float32),
        pltpu.VMEM((EC, d), jnp.float32),
        pltpu.SemaphoreType.DMA,
        pltpu.SemaphoreType.DMA,
        pltpu.SemaphoreType.DMA,
        pltpu.SemaphoreType.DMA,
    ]
    if with_cnt:
        out_type.append(jax.ShapeDtypeStruct((NC, NP), jnp.float32))
        scratch += [
            pltpu.VMEM_SHARED((NP,), jnp.float32),
            pltpu.VMEM((EC,), jnp.float32),
        ]

    def body(y_hbm, src_hbm, dst_hbm, *rest):
        if with_cnt:
            (out_hbm, cnt_hbm, acc, srcv, dstv, rows, rows1, sem, sem1,
             ssem, ssem1, accc, onesv) = rest
        else:
            (out_hbm, acc, srcv, dstv, rows, rows1, sem, sem1,
             ssem, ssem1) = rest
        cid = lax.axis_index("c")
        sid = lax.axis_index("s")
        wid = cid * NS + sid

        zero16 = jnp.zeros((16,), jnp.float32)

        def _zrow(i, _):
            for j in range(d // 16):
                rows[i, pl.ds(j * 16, 16)] = zero16
            return 0

        with jax.named_scope("sc_init"):
            lax.fori_loop(0, EC, _zrow, 0)
            for k in range(RPT // EC):
                pltpu.sync_copy(rows, acc.at[pl.ds(sid * RPT + k * EC, EC)])

        if with_cnt:
            for j in range(EC // 16):
                onesv[pl.ds(j * 16, 16)] = zero16
            for k in range(RPT // EC):
                pltpu.sync_copy(onesv, accc.at[pl.ds(sid * RPT + k * EC, EC)])
            one16 = jnp.ones((16,), jnp.float32)
            for j in range(EC // 16):
                onesv[pl.ds(j * 16, 16)] = one16

        plsc.subcore_barrier()

        npair = GR // 2

        def _group(g, _):
            base = wid * RW + g * GR
            pltpu.sync_copy(src_hbm.at[pl.ds(base, GR)], srcv)
            pltpu.sync_copy(dst_hbm.at[pl.ds(base, GR)], dstv)
            pltpu.async_copy(y_hbm.at[srcv.at[0]], rows, sem)

            def _pair(p, _):
                j0 = 2 * p
                pltpu.make_async_copy(y_hbm.at[srcv.at[j0]], rows, sem).wait()

                @pl.when(p > 0)
                def _():
                    pltpu.make_async_copy(
                        rows1, acc.at[dstv.at[j0 - 1]], ssem1).wait()

                pltpu.async_copy(y_hbm.at[srcv.at[j0 + 1]], rows1, sem1)
                pltpu.async_copy(rows, acc.at[dstv.at[j0]], ssem, add=True)
                if with_cnt:
                    pltpu.sync_copy(onesv, accc.at[dstv.at[j0]], add=True)
                pltpu.make_async_copy(
                    y_hbm.at[srcv.at[j0 + 1]], rows1, sem1).wait()
                pltpu.make_async_copy(rows, acc.at[dstv.at[j0]], ssem).wait()

                @pl.when(p < npair - 1)
                def _():
                    pltpu.async_copy(y_hbm.at[srcv.at[j0 + 2]], rows, sem)

                pltpu.async_copy(rows1, acc.at[dstv.at[j0 + 1]], ssem1,
                                 add=True)
                if with_cnt:
                    pltpu.sync_copy(onesv, accc.at[dstv.at[j0 + 1]], add=True)
                return 0

            lax.fori_loop(0, npair, _pair, 0)
            pltpu.make_async_copy(
                rows1, acc.at[dstv.at[GR - 1]], ssem1).wait()
            return 0

        with jax.named_scope("sc_edge_loop"):
            lax.fori_loop(0, RW // GR, _group, 0)

        with jax.named_scope("sc_wb"):
            plsc.subcore_barrier()
            pltpu.sync_copy(acc.at[pl.ds(sid * RPT, RPT)],
                            out_hbm.at[cid, pl.ds(sid * RPT, RPT)])
            if with_cnt:
                pltpu.sync_copy(accc.at[pl.ds(sid * RPT, RPT)],
                                cnt_hbm.at[cid, pl.ds(sid * RPT, RPT)])

    return pl.kernel(body, out_type=out_type, mesh=mesh, scratch_types=scratch)


_seg_sum_cnt_128 = _seg_sum_kernel(D_HID, True)
_seg_sum_128 = _seg_sum_kernel(D_HID, False)

_BR = 1000
_GRID = N // _BR


def _mm_body(x_ref, w_ref, o_ref):
    o_ref[...] = jnp.dot(x_ref[...], w_ref[...], preferred_element_type=jnp.float32)


def _tc_matmul(x, w):
    n, k = x.shape
    m = w.shape[1]
    return pl.pallas_call(
        _mm_body,
        grid=(_GRID,),
        in_specs=[pl.BlockSpec((_BR, k), lambda i: (i, 0)),
                  pl.BlockSpec((k, m), lambda i: (0, 0))],
        out_specs=pl.BlockSpec((_BR, m), lambda i: (i, 0)),
        out_shape=jax.ShapeDtypeStruct((n, m), jnp.float32),
    )(x, w)


def _comb_body(s_ref, c_ref, h_ref, wr_ref, b_ref, g_ref, be_ref, rm_ref, rv_ref,
               wn_ref, h_out, y_out):
    rc = 1.0 / jnp.maximum(c_ref[0] + c_ref[1], 1.0)
    agg = (s_ref[0] + s_ref[1]) * rc
    h = agg + jnp.dot(h_ref[...], wr_ref[...],
                      preferred_element_type=jnp.float32) + b_ref[...]
    scale = g_ref[...] * lax.rsqrt(rv_ref[...] + EPS)
    h = (h - rm_ref[...]) * scale + be_ref[...]
    h = jnp.maximum(h, 0.0)
    h_out[...] = h
    y_out[...] = jnp.dot(h, wn_ref[...], preferred_element_type=jnp.float32)


def _tc_combine(s, c, h, wr, b, g, be, rm, rv, wn):
    d = s.shape[2]
    dn = wn.shape[1]
    vec = lambda: pl.BlockSpec((1, d), lambda i: (0, 0))
    return pl.pallas_call(
        _comb_body,
        grid=(_GRID,),
        in_specs=[pl.BlockSpec((NC, _BR, d), lambda i: (0, i, 0)),
                  pl.BlockSpec((NC, _BR, 1), lambda i: (0, i, 0)),
                  pl.BlockSpec((_BR, d), lambda i: (i, 0)),
                  pl.BlockSpec((d, d), lambda i: (0, 0)),
                  vec(), vec(), vec(), vec(), vec(),
                  pl.BlockSpec((d, dn), lambda i: (0, 0))],
        out_specs=[pl.BlockSpec((_BR, d), lambda i: (i, 0)),
                   pl.BlockSpec((_BR, dn), lambda i: (i, 0))],
        out_shape=[jax.ShapeDtypeStruct((N, d), jnp.float32),
                   jax.ShapeDtypeStruct((N, dn), jnp.float32)],
    )(s, c, h, wr, b.reshape(1, d), g.reshape(1, d), be.reshape(1, d),
      rm.reshape(1, d), rv.reshape(1, d), wn)


def _fin_body(s_ref, c_ref, h_ref, wl_ref, wr_ref, b_ref, o_out):
    rc = 1.0 / jnp.maximum(c_ref[0] + c_ref[1], 1.0)
    agg = (s_ref[0] + s_ref[1]) * rc
    o = (jnp.dot(agg, wl_ref[...], preferred_element_type=jnp.float32)
         + jnp.dot(h_ref[...], wr_ref[...], preferred_element_type=jnp.float32)
         + b_ref[...])
    m = jnp.max(o, axis=1, keepdims=True)
    e = jnp.exp(o - m)
    o_out[...] = (o - m) - jnp.log(jnp.sum(e, axis=1, keepdims=True))


def _tc_final(s, c, h, wl, wr, b):
    d = h.shape[1]
    dn = wr.shape[1]
    return pl.pallas_call(
        _fin_body,
        grid=(_GRID,),
        in_specs=[pl.BlockSpec((NC, _BR, d), lambda i: (0, i, 0)),
                  pl.BlockSpec((NC, _BR, 1), lambda i: (0, i, 0)),
                  pl.BlockSpec((_BR, d), lambda i: (i, 0)),
                  pl.BlockSpec((d, dn), lambda i: (0, 0)),
                  pl.BlockSpec((d, dn), lambda i: (0, 0)),
                  pl.BlockSpec((1, dn), lambda i: (0, 0))],
        out_specs=pl.BlockSpec((_BR, dn), lambda i: (i, 0)),
        out_shape=jax.ShapeDtypeStruct((N, dn), jnp.float32),
    )(s, c, h, wl, wr, b.reshape(1, dn))


def kernel(x, edge_index, W0l, b0l, W0r, g0, be0, rm0, rv0,
           W1l, b1l, W1r, g1, be1, rm1, rv1, W2l, b2l, W2r):
    pad_i = jnp.arange(EP - E, dtype=jnp.int32)
    src = jnp.concatenate(
        [edge_index[0], pad_i % N]).reshape(EP // EC, EC)
    dst = jnp.concatenate(
        [edge_index[1], N + pad_i % (NP - N)]).reshape(EP // EC, EC)

    y0 = _tc_matmul(x, W0l)
    s0, cnt = _seg_sum_cnt_128(y0, src, dst)
    cnt = cnt.reshape(NC, NP, 1)
    h1, y1 = _tc_combine(s0, cnt, x, W0r, b0l, g0, be0, rm0, rv0, W1l)
    s1, = _seg_sum_128(y1, src, dst)
    h2, _ = _tc_combine(s1, cnt, h1, W1r, b1l, g1, be1, rm1, rv1, W2l)
    s2, = _seg_sum_128(h2, src, dst)
    return _tc_final(s2, cnt, h2, W2l, W2r, b2l)


# speedup vs baseline: 1.1208x; 1.1208x over previous
import functools

import jax
import jax.numpy as jnp
from jax import lax
from jax.experimental import pallas as pl
from jax.experimental.pallas import tpu as pltpu
from jax.experimental.pallas import tpu_sc as plsc

N = 10000
NP = 10240
E = 320000
D_IN = 128
D_HID = 128
D_OUT = 64
EPS = 1e-5

NC = 2
NS = 16
NW = NC * NS
EC = 96
EP = 344064
RW = EP // (NW * EC)
GR = 16
RPT = NP // NS


def _seg_sum_kernel(d, with_cnt):
    mesh = plsc.VectorSubcoreMesh(
        core_axis_name="c", subcore_axis_name="s", num_cores=NC, num_subcores=NS)

    out_type = [jax.ShapeDtypeStruct((NC, NP, d), jnp.float32)]
    scratch = [
        pltpu.VMEM_SHARED((NP, d), jnp.float32),
        pltpu.VMEM((GR, EC), jnp.int32),
        pltpu.VMEM((GR, EC), jnp.int32),
        pltpu.VMEM((EC, d), jnp.float32),
        pltpu.VMEM((EC, d), jnp.float32),
        pltpu.SemaphoreType.DMA,
        pltpu.SemaphoreType.DMA,
        pltpu.SemaphoreType.DMA,
        pltpu.SemaphoreType.DMA,
    ]
    if with_cnt:
        out_type.append(jax.ShapeDtypeStruct((NC, NP), jnp.float32))
        scratch += [
            pltpu.VMEM_SHARED((NP,), jnp.float32),
            pltpu.VMEM((EC,), jnp.float32),
        ]

    def body(y_hbm, src_hbm, dst_hbm, *rest):
        if with_cnt:
            (out_hbm, cnt_hbm, acc, srcv, dstv, rows, rows1, sem, sem1,
             ssem, ssem1, accc, onesv) = rest
        else:
            (out_hbm, acc, srcv, dstv, rows, rows1, sem, sem1,
             ssem, ssem1) = rest
        cid = lax.axis_index("c")
        sid = lax.axis_index("s")
        wid = cid * NS + sid

        zero16 = jnp.zeros((16,), jnp.float32)

        def _zrow(i, _):
            for j in range(d // 16):
                rows[i, pl.ds(j * 16, 16)] = zero16
            return 0

        rem = RPT % EC
        with jax.named_scope("sc_init"):
            lax.fori_loop(0, EC, _zrow, 0)
            for k in range(RPT // EC):
                pltpu.sync_copy(rows, acc.at[pl.ds(sid * RPT + k * EC, EC)])
            if rem:
                pltpu.sync_copy(
                    rows.at[pl.ds(0, rem)],
                    acc.at[pl.ds(sid * RPT + (RPT // EC) * EC, rem)])

        if with_cnt:
            for j in range(EC // 16):
                onesv[pl.ds(j * 16, 16)] = zero16
            for k in range(RPT // EC):
                pltpu.sync_copy(onesv, accc.at[pl.ds(sid * RPT + k * EC, EC)])
            if rem:
                pltpu.sync_copy(
                    onesv.at[pl.ds(0, rem)],
                    accc.at[pl.ds(sid * RPT + (RPT // EC) * EC, rem)])
            one16 = jnp.ones((16,), jnp.float32)
            for j in range(EC // 16):
                onesv[pl.ds(j * 16, 16)] = one16

        plsc.subcore_barrier()

        npair = GR // 2

        def _group(g, _):
            base = wid * RW + g * GR
            pltpu.sync_copy(src_hbm.at[pl.ds(base, GR)], srcv)
            pltpu.sync_copy(dst_hbm.at[pl.ds(base, GR)], dstv)
            pltpu.async_copy(y_hbm.at[srcv.at[0]], rows, sem)

            def _pair(p, _):
                j0 = 2 * p
                pltpu.make_async_copy(y_hbm.at[srcv.at[j0]], rows, sem).wait()

                @pl.when(p > 0)
                def _():
                    pltpu.make_async_copy(
                        rows1, acc.at[dstv.at[j0 - 1]], ssem1).wait()

                pltpu.async_copy(y_hbm.at[srcv.at[j0 + 1]], rows1, sem1)
                pltpu.async_copy(rows, acc.at[dstv.at[j0]], ssem, add=True)
                if with_cnt:
                    pltpu.sync_copy(onesv, accc.at[dstv.at[j0]], add=True)
                pltpu.make_async_copy(
                    y_hbm.at[srcv.at[j0 + 1]], rows1, sem1).wait()
                pltpu.make_async_copy(rows, acc.at[dstv.at[j0]], ssem).wait()

                @pl.when(p < npair - 1)
                def _():
                    pltpu.async_copy(y_hbm.at[srcv.at[j0 + 2]], rows, sem)

                pltpu.async_copy(rows1, acc.at[dstv.at[j0 + 1]], ssem1,
                                 add=True)
                if with_cnt:
                    pltpu.sync_copy(onesv, accc.at[dstv.at[j0 + 1]], add=True)
                return 0

            lax.fori_loop(0, npair, _pair, 0)
            pltpu.make_async_copy(
                rows1, acc.at[dstv.at[GR - 1]], ssem1).wait()
            return 0

        with jax.named_scope("sc_edge_loop"):
            lax.fori_loop(0, RW // GR, _group, 0)

        with jax.named_scope("sc_wb"):
            plsc.subcore_barrier()
            pltpu.sync_copy(acc.at[pl.ds(sid * RPT, RPT)],
                            out_hbm.at[cid, pl.ds(sid * RPT, RPT)])
            if with_cnt:
                pltpu.sync_copy(accc.at[pl.ds(sid * RPT, RPT)],
                                cnt_hbm.at[cid, pl.ds(sid * RPT, RPT)])

    return pl.kernel(body, out_type=out_type, mesh=mesh, scratch_types=scratch)


_seg_sum_cnt_128 = _seg_sum_kernel(D_HID, True)
_seg_sum_128 = _seg_sum_kernel(D_HID, False)

_BR = 1000
_GRID = N // _BR


def _mm_body(x_ref, w_ref, o_ref):
    o_ref[...] = jnp.dot(x_ref[...], w_ref[...], preferred_element_type=jnp.float32)


def _tc_matmul(x, w):
    n, k = x.shape
    m = w.shape[1]
    return pl.pallas_call(
        _mm_body,
        grid=(_GRID,),
        in_specs=[pl.BlockSpec((_BR, k), lambda i: (i, 0)),
                  pl.BlockSpec((k, m), lambda i: (0, 0))],
        out_specs=pl.BlockSpec((_BR, m), lambda i: (i, 0)),
        out_shape=jax.ShapeDtypeStruct((n, m), jnp.float32),
    )(x, w)


def _comb_body(s_ref, c_ref, h_ref, wr_ref, b_ref, g_ref, be_ref, rm_ref, rv_ref,
               wn_ref, h_out, y_out):
    rc = 1.0 / jnp.maximum(c_ref[0] + c_ref[1], 1.0)
    agg = (s_ref[0] + s_ref[1]) * rc
    h = agg + jnp.dot(h_ref[...], wr_ref[...],
                      preferred_element_type=jnp.float32) + b_ref[...]
    scale = g_ref[...] * lax.rsqrt(rv_ref[...] + EPS)
    h = (h - rm_ref[...]) * scale + be_ref[...]
    h = jnp.maximum(h, 0.0)
    h_out[...] = h
    y_out[...] = jnp.dot(h, wn_ref[...], preferred_element_type=jnp.float32)


def _tc_combine(s, c, h, wr, b, g, be, rm, rv, wn):
    d = s.shape[2]
    dn = wn.shape[1]
    vec = lambda: pl.BlockSpec((1, d), lambda i: (0, 0))
    return pl.pallas_call(
        _comb_body,
        grid=(_GRID,),
        in_specs=[pl.BlockSpec((NC, _BR, d), lambda i: (0, i, 0)),
                  pl.BlockSpec((NC, _BR, 1), lambda i: (0, i, 0)),
                  pl.BlockSpec((_BR, d), lambda i: (i, 0)),
                  pl.BlockSpec((d, d), lambda i: (0, 0)),
                  vec(), vec(), vec(), vec(), vec(),
                  pl.BlockSpec((d, dn), lambda i: (0, 0))],
        out_specs=[pl.BlockSpec((_BR, d), lambda i: (i, 0)),
                   pl.BlockSpec((_BR, dn), lambda i: (i, 0))],
        out_shape=[jax.ShapeDtypeStruct((N, d), jnp.float32),
                   jax.ShapeDtypeStruct((N, dn), jnp.float32)],
    )(s, c, h, wr, b.reshape(1, d), g.reshape(1, d), be.reshape(1, d),
      rm.reshape(1, d), rv.reshape(1, d), wn)


def _fin_body(s_ref, c_ref, h_ref, wl_ref, wr_ref, b_ref, o_out):
    rc = 1.0 / jnp.maximum(c_ref[0] + c_ref[1], 1.0)
    agg = (s_ref[0] + s_ref[1]) * rc
    o = (jnp.dot(agg, wl_ref[...], preferred_element_type=jnp.float32)
         + jnp.dot(h_ref[...], wr_ref[...], preferred_element_type=jnp.float32)
         + b_ref[...])
    m = jnp.max(o, axis=1, keepdims=True)
    e = jnp.exp(o - m)
    o_out[...] = (o - m) - jnp.log(jnp.sum(e, axis=1, keepdims=True))


def _tc_final(s, c, h, wl, wr, b):
    d = h.shape[1]
    dn = wr.shape[1]
    return pl.pallas_call(
        _fin_body,
        grid=(_GRID,),
        in_specs=[pl.BlockSpec((NC, _BR, d), lambda i: (0, i, 0)),
                  pl.BlockSpec((NC, _BR, 1), lambda i: (0, i, 0)),
                  pl.BlockSpec((_BR, d), lambda i: (i, 0)),
                  pl.BlockSpec((d, dn), lambda i: (0, 0)),
                  pl.BlockSpec((d, dn), lambda i: (0, 0)),
                  pl.BlockSpec((1, dn), lambda i: (0, 0))],
        out_specs=pl.BlockSpec((_BR, dn), lambda i: (i, 0)),
        out_shape=jax.ShapeDtypeStruct((N, dn), jnp.float32),
    )(s, c, h, wl, wr, b.reshape(1, dn))


def kernel(x, edge_index, W0l, b0l, W0r, g0, be0, rm0, rv0,
           W1l, b1l, W1r, g1, be1, rm1, rv1, W2l, b2l, W2r):
    pad_i = jnp.arange(EP - E, dtype=jnp.int32)
    src = jnp.concatenate(
        [edge_index[0], pad_i % N]).reshape(EP // EC, EC)
    dst = jnp.concatenate(
        [edge_index[1], N + pad_i % (NP - N)]).reshape(EP // EC, EC)

    y0 = _tc_matmul(x, W0l)
    s0, cnt = _seg_sum_cnt_128(y0, src, dst)
    cnt = cnt.reshape(NC, NP, 1)
    h1, y1 = _tc_combine(s0, cnt, x, W0r, b0l, g0, be0, rm0, rv0, W1l)
    s1, = _seg_sum_128(y1, src, dst)
    h2, _ = _tc_combine(s1, cnt, h1, W1r, b1l, g1, be1, rm1, rv1, W2l)
    s2, = _seg_sum_128(h2, src, dst)
    return _tc_final(s2, cnt, h2, W2l, W2r, b2l)


# speedup vs baseline: 1.2253x; 1.0933x over previous
import functools

import jax
import jax.numpy as jnp
from jax import lax
from jax.experimental import pallas as pl
from jax.experimental.pallas import tpu as pltpu
from jax.experimental.pallas import tpu_sc as plsc

N = 10000
NP = 10240
E = 320000
D_IN = 128
D_HID = 128
D_OUT = 64
EPS = 1e-5

NC = 2
NS = 16
NW = NC * NS
EC = 125
EP = 320000
RW = EP // (NW * EC)
GR = 8
RPT = NP // NS


def _seg_sum_kernel(d, with_cnt):
    mesh = plsc.VectorSubcoreMesh(
        core_axis_name="c", subcore_axis_name="s", num_cores=NC, num_subcores=NS)

    out_type = [jax.ShapeDtypeStruct((NC, NP, d), jnp.float32)]
    scratch = [
        pltpu.VMEM_SHARED((NP, d), jnp.float32),
        pltpu.VMEM((GR, EC), jnp.int32),
        pltpu.VMEM((GR, EC), jnp.int32),
        pltpu.VMEM((EC, d), jnp.float32),
        pltpu.VMEM((EC, d), jnp.float32),
        pltpu.SemaphoreType.DMA,
        pltpu.SemaphoreType.DMA,
        pltpu.SemaphoreType.DMA,
        pltpu.SemaphoreType.DMA,
    ]
    if with_cnt:
        out_type.append(jax.ShapeDtypeStruct((NC, NP), jnp.float32))
        scratch += [
            pltpu.VMEM_SHARED((NP,), jnp.float32),
            pltpu.VMEM((128,), jnp.float32),
        ]

    def body(y_hbm, src_hbm, dst_hbm, *rest):
        if with_cnt:
            (out_hbm, cnt_hbm, acc, srcv, dstv, rows, rows1, sem, sem1,
             ssem, ssem1, accc, onesv) = rest
        else:
            (out_hbm, acc, srcv, dstv, rows, rows1, sem, sem1,
             ssem, ssem1) = rest
        cid = lax.axis_index("c")
        sid = lax.axis_index("s")
        wid = cid * NS + sid

        zero16 = jnp.zeros((16,), jnp.float32)

        def _zrow(i, _):
            for j in range(d // 16):
                rows[i, pl.ds(j * 16, 16)] = zero16
            return 0

        ZC = 80
        with jax.named_scope("sc_init"):
            lax.fori_loop(0, EC, _zrow, 0)
            for k in range(RPT // ZC):
                pltpu.sync_copy(rows.at[pl.ds(0, ZC)],
                                acc.at[pl.ds(sid * RPT + k * ZC, ZC)])

        if with_cnt:
            for j in range(8):
                onesv[pl.ds(j * 16, 16)] = zero16
            for k in range(RPT // ZC):
                pltpu.sync_copy(onesv.at[pl.ds(0, ZC)],
                                accc.at[pl.ds(sid * RPT + k * ZC, ZC)])
            one16 = jnp.ones((16,), jnp.float32)
            for j in range(8):
                onesv[pl.ds(j * 16, 16)] = one16

        plsc.subcore_barrier()

        npair = GR // 2

        def _group(g, _):
            base = wid * RW + g * GR
            pltpu.sync_copy(src_hbm.at[pl.ds(base, GR)], srcv)
            pltpu.sync_copy(dst_hbm.at[pl.ds(base, GR)], dstv)
            pltpu.async_copy(y_hbm.at[srcv.at[0]], rows, sem)

            def _pair(p, _):
                j0 = 2 * p
                pltpu.make_async_copy(y_hbm.at[srcv.at[j0]], rows, sem).wait()

                @pl.when(p > 0)
                def _():
                    pltpu.make_async_copy(
                        rows1, acc.at[dstv.at[j0 - 1]], ssem1).wait()

                pltpu.async_copy(y_hbm.at[srcv.at[j0 + 1]], rows1, sem1)
                pltpu.async_copy(rows, acc.at[dstv.at[j0]], ssem, add=True)
                if with_cnt:
                    pltpu.sync_copy(onesv.at[pl.ds(0, EC)],
                                    accc.at[dstv.at[j0]], add=True)
                pltpu.make_async_copy(
                    y_hbm.at[srcv.at[j0 + 1]], rows1, sem1).wait()
                pltpu.make_async_copy(rows, acc.at[dstv.at[j0]], ssem).wait()

                @pl.when(p < npair - 1)
                def _():
                    pltpu.async_copy(y_hbm.at[srcv.at[j0 + 2]], rows, sem)

                pltpu.async_copy(rows1, acc.at[dstv.at[j0 + 1]], ssem1,
                                 add=True)
                if with_cnt:
                    pltpu.sync_copy(onesv.at[pl.ds(0, EC)],
                                    accc.at[dstv.at[j0 + 1]], add=True)
                return 0

            lax.fori_loop(0, npair, _pair, 0)
            pltpu.make_async_copy(
                rows1, acc.at[dstv.at[GR - 1]], ssem1).wait()
            return 0

        with jax.named_scope("sc_edge_loop"):
            lax.fori_loop(0, RW // GR, _group, 0)

        with jax.named_scope("sc_wb"):
            plsc.subcore_barrier()
            pltpu.sync_copy(acc.at[pl.ds(sid * RPT, RPT)],
                            out_hbm.at[cid, pl.ds(sid * RPT, RPT)])
            if with_cnt:
                pltpu.sync_copy(accc.at[pl.ds(sid * RPT, RPT)],
                                cnt_hbm.at[cid, pl.ds(sid * RPT, RPT)])

    return pl.kernel(body, out_type=out_type, mesh=mesh, scratch_types=scratch)


_seg_sum_cnt_128 = _seg_sum_kernel(D_HID, True)
_seg_sum_128 = _seg_sum_kernel(D_HID, False)

_BR = 1000
_GRID = N // _BR


def _mm_body(x_ref, w_ref, o_ref):
    o_ref[...] = jnp.dot(x_ref[...], w_ref[...], preferred_element_type=jnp.float32)


def _tc_matmul(x, w):
    n, k = x.shape
    m = w.shape[1]
    return pl.pallas_call(
        _mm_body,
        grid=(_GRID,),
        in_specs=[pl.BlockSpec((_BR, k), lambda i: (i, 0)),
                  pl.BlockSpec((k, m), lambda i: (0, 0))],
        out_specs=pl.BlockSpec((_BR, m), lambda i: (i, 0)),
        out_shape=jax.ShapeDtypeStruct((n, m), jnp.float32),
    )(x, w)


def _comb_body(s_ref, c_ref, h_ref, wr_ref, b_ref, g_ref, be_ref, rm_ref, rv_ref,
               wn_ref, h_out, y_out):
    rc = 1.0 / jnp.maximum(c_ref[0] + c_ref[1], 1.0)
    agg = (s_ref[0] + s_ref[1]) * rc
    h = agg + jnp.dot(h_ref[...], wr_ref[...],
                      preferred_element_type=jnp.float32) + b_ref[...]
    scale = g_ref[...] * lax.rsqrt(rv_ref[...] + EPS)
    h = (h - rm_ref[...]) * scale + be_ref[...]
    h = jnp.maximum(h, 0.0)
    h_out[...] = h
    y_out[...] = jnp.dot(h, wn_ref[...], preferred_element_type=jnp.float32)


def _tc_combine(s, c, h, wr, b, g, be, rm, rv, wn):
    d = s.shape[2]
    dn = wn.shape[1]
    vec = lambda: pl.BlockSpec((1, d), lambda i: (0, 0))
    return pl.pallas_call(
        _comb_body,
        grid=(_GRID,),
        in_specs=[pl.BlockSpec((NC, _BR, d), lambda i: (0, i, 0)),
                  pl.BlockSpec((NC, _BR, 1), lambda i: (0, i, 0)),
                  pl.BlockSpec((_BR, d), lambda i: (i, 0)),
                  pl.BlockSpec((d, d), lambda i: (0, 0)),
                  vec(), vec(), vec(), vec(), vec(),
                  pl.BlockSpec((d, dn), lambda i: (0, 0))],
        out_specs=[pl.BlockSpec((_BR, d), lambda i: (i, 0)),
                   pl.BlockSpec((_BR, dn), lambda i: (i, 0))],
        out_shape=[jax.ShapeDtypeStruct((N, d), jnp.float32),
                   jax.ShapeDtypeStruct((N, dn), jnp.float32)],
    )(s, c, h, wr, b.reshape(1, d), g.reshape(1, d), be.reshape(1, d),
      rm.reshape(1, d), rv.reshape(1, d), wn)


def _fin_body(s_ref, c_ref, h_ref, wl_ref, wr_ref, b_ref, o_out):
    rc = 1.0 / jnp.maximum(c_ref[0] + c_ref[1], 1.0)
    agg = (s_ref[0] + s_ref[1]) * rc
    o = (jnp.dot(agg, wl_ref[...], preferred_element_type=jnp.float32)
         + jnp.dot(h_ref[...], wr_ref[...], preferred_element_type=jnp.float32)
         + b_ref[...])
    m = jnp.max(o, axis=1, keepdims=True)
    e = jnp.exp(o - m)
    o_out[...] = (o - m) - jnp.log(jnp.sum(e, axis=1, keepdims=True))


def _tc_final(s, c, h, wl, wr, b):
    d = h.shape[1]
    dn = wr.shape[1]
    return pl.pallas_call(
        _fin_body,
        grid=(_GRID,),
        in_specs=[pl.BlockSpec((NC, _BR, d), lambda i: (0, i, 0)),
                  pl.BlockSpec((NC, _BR, 1), lambda i: (0, i, 0)),
                  pl.BlockSpec((_BR, d), lambda i: (i, 0)),
                  pl.BlockSpec((d, dn), lambda i: (0, 0)),
                  pl.BlockSpec((d, dn), lambda i: (0, 0)),
                  pl.BlockSpec((1, dn), lambda i: (0, 0))],
        out_specs=pl.BlockSpec((_BR, dn), lambda i: (i, 0)),
        out_shape=jax.ShapeDtypeStruct((N, dn), jnp.float32),
    )(s, c, h, wl, wr, b.reshape(1, dn))


def kernel(x, edge_index, W0l, b0l, W0r, g0, be0, rm0, rv0,
           W1l, b1l, W1r, g1, be1, rm1, rv1, W2l, b2l, W2r):
    src = edge_index[0].reshape(EP // EC, EC)
    dst = edge_index[1].reshape(EP // EC, EC)

    y0 = _tc_matmul(x, W0l)
    s0, cnt = _seg_sum_cnt_128(y0, src, dst)
    cnt = cnt.reshape(NC, NP, 1)
    h1, y1 = _tc_combine(s0, cnt, x, W0r, b0l, g0, be0, rm0, rv0, W1l)
    s1, = _seg_sum_128(y1, src, dst)
    h2, _ = _tc_combine(s1, cnt, h1, W1r, b1l, g1, be1, rm1, rv1, W2l)
    s2, = _seg_sum_128(h2, src, dst)
    return _tc_final(s2, cnt, h2, W2l, W2r, b2l)


# speedup vs baseline: 1.2767x; 1.0419x over previous
import functools

import jax
import jax.numpy as jnp
from jax import lax
from jax.experimental import pallas as pl
from jax.experimental.pallas import tpu as pltpu
from jax.experimental.pallas import tpu_sc as plsc

N = 10000
NP = 10240
E = 320000
D_IN = 128
D_HID = 128
D_OUT = 64
EPS = 1e-5

NC = 2
NS = 16
NW = NC * NS
EC = 125
EP = 320000
RW = EP // (NW * EC)
GR = 16
RPT = NP // NS


def _seg_sum_kernel(d, with_cnt):
    mesh = plsc.VectorSubcoreMesh(
        core_axis_name="c", subcore_axis_name="s", num_cores=NC, num_subcores=NS)

    out_type = [jax.ShapeDtypeStruct((NC, NP, d), jnp.float32)]
    scratch = [
        pltpu.VMEM_SHARED((NP, d), jnp.float32),
        pltpu.VMEM((GR, EC), jnp.int32),
        pltpu.VMEM((GR, EC), jnp.int32),
        pltpu.VMEM((EC, d), jnp.float32),
        pltpu.VMEM((EC, d), jnp.float32),
        pltpu.SemaphoreType.DMA,
        pltpu.SemaphoreType.DMA,
        pltpu.SemaphoreType.DMA,
        pltpu.SemaphoreType.DMA,
    ]
    if with_cnt:
        out_type.append(jax.ShapeDtypeStruct((NC, NP), jnp.float32))
        scratch += [
            pltpu.VMEM_SHARED((NP,), jnp.float32),
            pltpu.VMEM((128,), jnp.float32),
        ]

    def body(y_hbm, src_hbm, dst_hbm, *rest):
        if with_cnt:
            (out_hbm, cnt_hbm, acc, srcv, dstv, rows, rows1, sem, sem1,
             ssem, ssem1, accc, onesv) = rest
        else:
            (out_hbm, acc, srcv, dstv, rows, rows1, sem, sem1,
             ssem, ssem1) = rest
        cid = lax.axis_index("c")
        sid = lax.axis_index("s")
        wid = cid * NS + sid

        zero16 = jnp.zeros((16,), jnp.float32)

        def _zrow(i, _):
            for j in range(d // 16):
                rows[i, pl.ds(j * 16, 16)] = zero16
            return 0

        ZC = 80
        with jax.named_scope("sc_init"):
            lax.fori_loop(0, EC, _zrow, 0)
            for k in range(RPT // ZC):
                pltpu.sync_copy(rows.at[pl.ds(0, ZC)],
                                acc.at[pl.ds(sid * RPT + k * ZC, ZC)])

        if with_cnt:
            for j in range(8):
                onesv[pl.ds(j * 16, 16)] = zero16
            for k in range(RPT // ZC):
                pltpu.sync_copy(onesv.at[pl.ds(0, ZC)],
                                accc.at[pl.ds(sid * RPT + k * ZC, ZC)])
            one16 = jnp.ones((16,), jnp.float32)
            for j in range(8):
                onesv[pl.ds(j * 16, 16)] = one16

        plsc.subcore_barrier()

        npair = GR // 2

        def _group(g, _):
            base = wid * RW + g * GR
            pltpu.sync_copy(src_hbm.at[pl.ds(base, GR)], srcv)
            pltpu.sync_copy(dst_hbm.at[pl.ds(base, GR)], dstv)
            pltpu.async_copy(y_hbm.at[srcv.at[0]], rows, sem)

            def _pair(p, _):
                j0 = 2 * p
                pltpu.make_async_copy(y_hbm.at[srcv.at[j0]], rows, sem).wait()

                @pl.when(p > 0)
                def _():
                    pltpu.make_async_copy(
                        rows1, acc.at[dstv.at[j0 - 1]], ssem1).wait()

                pltpu.async_copy(y_hbm.at[srcv.at[j0 + 1]], rows1, sem1)
                pltpu.async_copy(rows, acc.at[dstv.at[j0]], ssem, add=True)
                if with_cnt:
                    pltpu.sync_copy(onesv.at[pl.ds(0, EC)],
                                    accc.at[dstv.at[j0]], add=True)
                pltpu.make_async_copy(
                    y_hbm.at[srcv.at[j0 + 1]], rows1, sem1).wait()
                pltpu.make_async_copy(rows, acc.at[dstv.at[j0]], ssem).wait()

                @pl.when(p < npair - 1)
                def _():
                    pltpu.async_copy(y_hbm.at[srcv.at[j0 + 2]], rows, sem)

                pltpu.async_copy(rows1, acc.at[dstv.at[j0 + 1]], ssem1,
                                 add=True)
                if with_cnt:
                    pltpu.sync_copy(onesv.at[pl.ds(0, EC)],
                                    accc.at[dstv.at[j0 + 1]], add=True)
                return 0

            lax.fori_loop(0, npair, _pair, 0)
            pltpu.make_async_copy(
                rows1, acc.at[dstv.at[GR - 1]], ssem1).wait()
            return 0

        with jax.named_scope("sc_edge_loop"):
            lax.fori_loop(0, RW // GR, _group, 0)

        with jax.named_scope("sc_wb"):
            plsc.subcore_barrier()
            pltpu.sync_copy(acc.at[pl.ds(sid * RPT, RPT)],
                            out_hbm.at[cid, pl.ds(sid * RPT, RPT)])
            if with_cnt:
                pltpu.sync_copy(accc.at[pl.ds(sid * RPT, RPT)],
                                cnt_hbm.at[cid, pl.ds(sid * RPT, RPT)])

    return pl.kernel(body, out_type=out_type, mesh=mesh, scratch_types=scratch)


_seg_sum_cnt_128 = _seg_sum_kernel(D_HID, True)
_seg_sum_128 = _seg_sum_kernel(D_HID, False)

_BR = 1000
_GRID = N // _BR


def _mm_body(x_ref, w_ref, o_ref):
    o_ref[...] = jnp.dot(x_ref[...], w_ref[...], preferred_element_type=jnp.float32)


def _tc_matmul(x, w):
    n, k = x.shape
    m = w.shape[1]
    return pl.pallas_call(
        _mm_body,
        grid=(_GRID,),
        in_specs=[pl.BlockSpec((_BR, k), lambda i: (i, 0)),
                  pl.BlockSpec((k, m), lambda i: (0, 0))],
        out_specs=pl.BlockSpec((_BR, m), lambda i: (i, 0)),
        out_shape=jax.ShapeDtypeStruct((n, m), jnp.float32),
    )(x, w)


def _comb_body(s_ref, c_ref, h_ref, wr_ref, b_ref, g_ref, be_ref, rm_ref, rv_ref,
               wn_ref, h_out, y_out):
    rc = 1.0 / jnp.maximum(c_ref[0] + c_ref[1], 1.0)
    agg = (s_ref[0] + s_ref[1]) * rc
    h = agg + jnp.dot(h_ref[...], wr_ref[...],
                      preferred_element_type=jnp.float32) + b_ref[...]
    scale = g_ref[...] * lax.rsqrt(rv_ref[...] + EPS)
    h = (h - rm_ref[...]) * scale + be_ref[...]
    h = jnp.maximum(h, 0.0)
    h_out[...] = h
    y_out[...] = jnp.dot(h, wn_ref[...], preferred_element_type=jnp.float32)


def _tc_combine(s, c, h, wr, b, g, be, rm, rv, wn):
    d = s.shape[2]
    dn = wn.shape[1]
    vec = lambda: pl.BlockSpec((1, d), lambda i: (0, 0))
    return pl.pallas_call(
        _comb_body,
        grid=(_GRID,),
        in_specs=[pl.BlockSpec((NC, _BR, d), lambda i: (0, i, 0)),
                  pl.BlockSpec((NC, _BR, 1), lambda i: (0, i, 0)),
                  pl.BlockSpec((_BR, d), lambda i: (i, 0)),
                  pl.BlockSpec((d, d), lambda i: (0, 0)),
                  vec(), vec(), vec(), vec(), vec(),
                  pl.BlockSpec((d, dn), lambda i: (0, 0))],
        out_specs=[pl.BlockSpec((_BR, d), lambda i: (i, 0)),
                   pl.BlockSpec((_BR, dn), lambda i: (i, 0))],
        out_shape=[jax.ShapeDtypeStruct((N, d), jnp.float32),
                   jax.ShapeDtypeStruct((N, dn), jnp.float32)],
    )(s, c, h, wr, b.reshape(1, d), g.reshape(1, d), be.reshape(1, d),
      rm.reshape(1, d), rv.reshape(1, d), wn)


def _fin_body(s_ref, c_ref, h_ref, wl_ref, wr_ref, b_ref, o_out):
    rc = 1.0 / jnp.maximum(c_ref[0] + c_ref[1], 1.0)
    agg = (s_ref[0] + s_ref[1]) * rc
    o = (jnp.dot(agg, wl_ref[...], preferred_element_type=jnp.float32)
         + jnp.dot(h_ref[...], wr_ref[...], preferred_element_type=jnp.float32)
         + b_ref[...])
    m = jnp.max(o, axis=1, keepdims=True)
    e = jnp.exp(o - m)
    o_out[...] = (o - m) - jnp.log(jnp.sum(e, axis=1, keepdims=True))


def _tc_final(s, c, h, wl, wr, b):
    d = h.shape[1]
    dn = wr.shape[1]
    return pl.pallas_call(
        _fin_body,
        grid=(_GRID,),
        in_specs=[pl.BlockSpec((NC, _BR, d), lambda i: (0, i, 0)),
                  pl.BlockSpec((NC, _BR, 1), lambda i: (0, i, 0)),
                  pl.BlockSpec((_BR, d), lambda i: (i, 0)),
                  pl.BlockSpec((d, dn), lambda i: (0, 0)),
                  pl.BlockSpec((d, dn), lambda i: (0, 0)),
                  pl.BlockSpec((1, dn), lambda i: (0, 0))],
        out_specs=pl.BlockSpec((_BR, dn), lambda i: (i, 0)),
        out_shape=jax.ShapeDtypeStruct((N, dn), jnp.float32),
    )(s, c, h, wl, wr, b.reshape(1, dn))


def kernel(x, edge_index, W0l, b0l, W0r, g0, be0, rm0, rv0,
           W1l, b1l, W1r, g1, be1, rm1, rv1, W2l, b2l, W2r):
    src = edge_index[0].reshape(EP // EC, EC)
    dst = edge_index[1].reshape(EP // EC, EC)

    y0 = _tc_matmul(x, W0l)
    s0, cnt = _seg_sum_cnt_128(y0, src, dst)
    cnt = cnt.reshape(NC, NP, 1)
    h1, y1 = _tc_combine(s0, cnt, x, W0r, b0l, g0, be0, rm0, rv0, W1l)
    s1, = _seg_sum_128(y1, src, dst)
    h2, _ = _tc_combine(s1, cnt, h1, W1r, b1l, g1, be1, rm1, rv1, W2l)
    s2, = _seg_sum_128(h2, src, dst)
    return _tc_final(s2, cnt, h2, W2l, W2r, b2l)


# speedup vs baseline: 1.3196x; 1.0336x over previous
import functools

import jax
import jax.numpy as jnp
from jax import lax
from jax.experimental import pallas as pl
from jax.experimental.pallas import tpu as pltpu
from jax.experimental.pallas import tpu_sc as plsc

N = 10000
NP = 10240
E = 320000
D_IN = 128
D_HID = 128
D_OUT = 64
EPS = 1e-5

NC = 2
NS = 16
NW = NC * NS
EC = 125
EP = 320000
RW = EP // (NW * EC)
GR = 40
RPT = NP // NS


def _seg_sum_kernel(d, with_cnt):
    mesh = plsc.VectorSubcoreMesh(
        core_axis_name="c", subcore_axis_name="s", num_cores=NC, num_subcores=NS)

    out_type = [jax.ShapeDtypeStruct((NC, NP, d), jnp.float32)]
    scratch = [
        pltpu.VMEM_SHARED((NP, d), jnp.float32),
        pltpu.VMEM((GR, EC), jnp.int32),
        pltpu.VMEM((GR, EC), jnp.int32),
        pltpu.VMEM((EC, d), jnp.float32),
        pltpu.VMEM((EC, d), jnp.float32),
        pltpu.SemaphoreType.DMA,
        pltpu.SemaphoreType.DMA,
        pltpu.SemaphoreType.DMA,
        pltpu.SemaphoreType.DMA,
    ]
    if with_cnt:
        out_type.append(jax.ShapeDtypeStruct((NC, NP), jnp.float32))
        scratch += [
            pltpu.VMEM_SHARED((NP,), jnp.float32),
            pltpu.VMEM((128,), jnp.float32),
        ]

    def body(y_hbm, src_hbm, dst_hbm, *rest):
        if with_cnt:
            (out_hbm, cnt_hbm, acc, srcv, dstv, rows, rows1, sem, sem1,
             ssem, ssem1, accc, onesv) = rest
        else:
            (out_hbm, acc, srcv, dstv, rows, rows1, sem, sem1,
             ssem, ssem1) = rest
        cid = lax.axis_index("c")
        sid = lax.axis_index("s")
        wid = cid * NS + sid

        zero16 = jnp.zeros((16,), jnp.float32)

        def _zrow(i, _):
            for j in range(d // 16):
                rows[i, pl.ds(j * 16, 16)] = zero16
            return 0

        ZC = 80
        with jax.named_scope("sc_init"):
            lax.fori_loop(0, EC, _zrow, 0)
            for k in range(RPT // ZC):
                pltpu.sync_copy(rows.at[pl.ds(0, ZC)],
                                acc.at[pl.ds(sid * RPT + k * ZC, ZC)])

        if with_cnt:
            for j in range(8):
                onesv[pl.ds(j * 16, 16)] = zero16
            for k in range(RPT // ZC):
                pltpu.sync_copy(onesv.at[pl.ds(0, ZC)],
                                accc.at[pl.ds(sid * RPT + k * ZC, ZC)])
            one16 = jnp.ones((16,), jnp.float32)
            for j in range(8):
                onesv[pl.ds(j * 16, 16)] = one16

        plsc.subcore_barrier()

        npair = GR // 2

        def _group(g, _):
            base = wid * RW + g * GR
            pltpu.sync_copy(src_hbm.at[pl.ds(base, GR)], srcv)
            pltpu.sync_copy(dst_hbm.at[pl.ds(base, GR)], dstv)
            pltpu.async_copy(y_hbm.at[srcv.at[0]], rows, sem)

            def _pair(p, _):
                j0 = 2 * p
                pltpu.make_async_copy(y_hbm.at[srcv.at[j0]], rows, sem).wait()

                @pl.when(p > 0)
                def _():
                    pltpu.make_async_copy(
                        rows1, acc.at[dstv.at[j0 - 1]], ssem1).wait()

                pltpu.async_copy(y_hbm.at[srcv.at[j0 + 1]], rows1, sem1)
                pltpu.async_copy(rows, acc.at[dstv.at[j0]], ssem, add=True)
                if with_cnt:
                    pltpu.sync_copy(onesv.at[pl.ds(0, EC)],
                                    accc.at[dstv.at[j0]], add=True)
                pltpu.make_async_copy(
                    y_hbm.at[srcv.at[j0 + 1]], rows1, sem1).wait()
                pltpu.make_async_copy(rows, acc.at[dstv.at[j0]], ssem).wait()

                @pl.when(p < npair - 1)
                def _():
                    pltpu.async_copy(y_hbm.at[srcv.at[j0 + 2]], rows, sem)

                pltpu.async_copy(rows1, acc.at[dstv.at[j0 + 1]], ssem1,
                                 add=True)
                if with_cnt:
                    pltpu.sync_copy(onesv.at[pl.ds(0, EC)],
                                    accc.at[dstv.at[j0 + 1]], add=True)
                return 0

            lax.fori_loop(0, npair, _pair, 0)
            pltpu.make_async_copy(
                rows1, acc.at[dstv.at[GR - 1]], ssem1).wait()
            return 0

        with jax.named_scope("sc_edge_loop"):
            lax.fori_loop(0, RW // GR, _group, 0)

        with jax.named_scope("sc_wb"):
            plsc.subcore_barrier()
            pltpu.sync_copy(acc.at[pl.ds(sid * RPT, RPT)],
                            out_hbm.at[cid, pl.ds(sid * RPT, RPT)])
            if with_cnt:
                pltpu.sync_copy(accc.at[pl.ds(sid * RPT, RPT)],
                                cnt_hbm.at[cid, pl.ds(sid * RPT, RPT)])

    return pl.kernel(body, out_type=out_type, mesh=mesh, scratch_types=scratch)


_seg_sum_cnt_128 = _seg_sum_kernel(D_HID, True)
_seg_sum_128 = _seg_sum_kernel(D_HID, False)

_BR = 1000
_GRID = N // _BR


def _mm_body(x_ref, w_ref, o_ref):
    o_ref[...] = jnp.dot(x_ref[...], w_ref[...], preferred_element_type=jnp.float32)


def _tc_matmul(x, w):
    n, k = x.shape
    m = w.shape[1]
    return pl.pallas_call(
        _mm_body,
        grid=(_GRID,),
        in_specs=[pl.BlockSpec((_BR, k), lambda i: (i, 0)),
                  pl.BlockSpec((k, m), lambda i: (0, 0))],
        out_specs=pl.BlockSpec((_BR, m), lambda i: (i, 0)),
        out_shape=jax.ShapeDtypeStruct((n, m), jnp.float32),
    )(x, w)


def _comb_body(s_ref, c_ref, h_ref, wr_ref, b_ref, g_ref, be_ref, rm_ref, rv_ref,
               wn_ref, h_out, y_out):
    rc = 1.0 / jnp.maximum(c_ref[0] + c_ref[1], 1.0)
    agg = (s_ref[0] + s_ref[1]) * rc
    h = agg + jnp.dot(h_ref[...], wr_ref[...],
                      preferred_element_type=jnp.float32) + b_ref[...]
    scale = g_ref[...] * lax.rsqrt(rv_ref[...] + EPS)
    h = (h - rm_ref[...]) * scale + be_ref[...]
    h = jnp.maximum(h, 0.0)
    h_out[...] = h
    y_out[...] = jnp.dot(h, wn_ref[...], preferred_element_type=jnp.float32)


def _tc_combine(s, c, h, wr, b, g, be, rm, rv, wn):
    d = s.shape[2]
    dn = wn.shape[1]
    vec = lambda: pl.BlockSpec((1, d), lambda i: (0, 0))
    return pl.pallas_call(
        _comb_body,
        grid=(_GRID,),
        in_specs=[pl.BlockSpec((NC, _BR, d), lambda i: (0, i, 0)),
                  pl.BlockSpec((NC, _BR, 1), lambda i: (0, i, 0)),
                  pl.BlockSpec((_BR, d), lambda i: (i, 0)),
                  pl.BlockSpec((d, d), lambda i: (0, 0)),
                  vec(), vec(), vec(), vec(), vec(),
                  pl.BlockSpec((d, dn), lambda i: (0, 0))],
        out_specs=[pl.BlockSpec((_BR, d), lambda i: (i, 0)),
                   pl.BlockSpec((_BR, dn), lambda i: (i, 0))],
        out_shape=[jax.ShapeDtypeStruct((N, d), jnp.float32),
                   jax.ShapeDtypeStruct((N, dn), jnp.float32)],
    )(s, c, h, wr, b.reshape(1, d), g.reshape(1, d), be.reshape(1, d),
      rm.reshape(1, d), rv.reshape(1, d), wn)


def _fin_body(s_ref, c_ref, h_ref, wl_ref, wr_ref, b_ref, o_out):
    rc = 1.0 / jnp.maximum(c_ref[0] + c_ref[1], 1.0)
    agg = (s_ref[0] + s_ref[1]) * rc
    o = (jnp.dot(agg, wl_ref[...], preferred_element_type=jnp.float32)
         + jnp.dot(h_ref[...], wr_ref[...], preferred_element_type=jnp.float32)
         + b_ref[...])
    m = jnp.max(o, axis=1, keepdims=True)
    e = jnp.exp(o - m)
    o_out[...] = (o - m) - jnp.log(jnp.sum(e, axis=1, keepdims=True))


def _tc_final(s, c, h, wl, wr, b):
    d = h.shape[1]
    dn = wr.shape[1]
    return pl.pallas_call(
        _fin_body,
        grid=(_GRID,),
        in_specs=[pl.BlockSpec((NC, _BR, d), lambda i: (0, i, 0)),
                  pl.BlockSpec((NC, _BR, 1), lambda i: (0, i, 0)),
                  pl.BlockSpec((_BR, d), lambda i: (i, 0)),
                  pl.BlockSpec((d, dn), lambda i: (0, 0)),
                  pl.BlockSpec((d, dn), lambda i: (0, 0)),
                  pl.BlockSpec((1, dn), lambda i: (0, 0))],
        out_specs=pl.BlockSpec((_BR, dn), lambda i: (i, 0)),
        out_shape=jax.ShapeDtypeStruct((N, dn), jnp.float32),
    )(s, c, h, wl, wr, b.reshape(1, dn))


def kernel(x, edge_index, W0l, b0l, W0r, g0, be0, rm0, rv0,
           W1l, b1l, W1r, g1, be1, rm1, rv1, W2l, b2l, W2r):
    src = edge_index[0].reshape(EP // EC, EC)
    dst = edge_index[1].reshape(EP // EC, EC)

    y0 = _tc_matmul(x, W0l)
    s0, cnt = _seg_sum_cnt_128(y0, src, dst)
    cnt = cnt.reshape(NC, NP, 1)
    h1, y1 = _tc_combine(s0, cnt, x, W0r, b0l, g0, be0, rm0, rv0, W1l)
    s1, = _seg_sum_128(y1, src, dst)
    h2, _ = _tc_combine(s1, cnt, h1, W1r, b1l, g1, be1, rm1, rv1, W2l)
    s2, = _seg_sum_128(h2, src, dst)
    return _tc_final(s2, cnt, h2, W2l, W2r, b2l)


# speedup vs baseline: 1.3310x; 1.0087x over previous
import functools

import jax
import jax.numpy as jnp
from jax import lax
from jax.experimental import pallas as pl
from jax.experimental.pallas import tpu as pltpu
from jax.experimental.pallas import tpu_sc as plsc

N = 10000
NP = 10240
E = 320000
D_IN = 128
D_HID = 128
D_OUT = 64
EPS = 1e-5

NC = 2
NS = 16
NW = NC * NS
EC = 125
EP = 320000
RW = EP // (NW * EC)
GR = 40
RPT = NP // NS


def _seg_sum_kernel(d, with_cnt):
    mesh = plsc.VectorSubcoreMesh(
        core_axis_name="c", subcore_axis_name="s", num_cores=NC, num_subcores=NS)

    out_type = [jax.ShapeDtypeStruct((NC, NP, d), jnp.float32)]
    scratch = [
        pltpu.VMEM_SHARED((NP, d), jnp.float32),
        pltpu.VMEM((GR, EC), jnp.int32),
        pltpu.VMEM((GR, EC), jnp.int32),
        pltpu.VMEM((EC, d), jnp.float32),
        pltpu.VMEM((EC, d), jnp.float32),
        pltpu.SemaphoreType.DMA,
        pltpu.SemaphoreType.DMA,
        pltpu.SemaphoreType.DMA,
        pltpu.SemaphoreType.DMA,
    ]
    if with_cnt:
        out_type.append(jax.ShapeDtypeStruct((NC, NP), jnp.float32))
        scratch += [
            pltpu.VMEM_SHARED((NP,), jnp.float32),
            pltpu.VMEM((128,), jnp.float32),
        ]

    def body(y_hbm, src_hbm, dst_hbm, *rest):
        if with_cnt:
            (out_hbm, cnt_hbm, acc, srcv, dstv, rows, rows1, sem, sem1,
             ssem, ssem1, accc, onesv) = rest
        else:
            (out_hbm, acc, srcv, dstv, rows, rows1, sem, sem1,
             ssem, ssem1) = rest
        cid = lax.axis_index("c")
        sid = lax.axis_index("s")
        wid = cid * NS + sid

        zero16 = jnp.zeros((16,), jnp.float32)

        def _zrow(i, _):
            for j in range(d // 16):
                rows[i, pl.ds(j * 16, 16)] = zero16
            return 0

        ZC = 80
        with jax.named_scope("sc_init"):
            lax.fori_loop(0, EC, _zrow, 0)
            for k in range(RPT // ZC):
                pltpu.sync_copy(rows.at[pl.ds(0, ZC)],
                                acc.at[pl.ds(sid * RPT + k * ZC, ZC)])

        if with_cnt:
            for j in range(8):
                onesv[pl.ds(j * 16, 16)] = zero16
            for k in range(RPT // ZC):
                pltpu.sync_copy(onesv.at[pl.ds(0, ZC)],
                                accc.at[pl.ds(sid * RPT + k * ZC, ZC)])
            one16 = jnp.ones((16,), jnp.float32)
            for j in range(8):
                onesv[pl.ds(j * 16, 16)] = one16

        plsc.subcore_barrier()

        npair = GR // 2

        def _group(g, _):
            base = wid * RW + g * GR
            pltpu.sync_copy(src_hbm.at[pl.ds(base, GR)], srcv)
            pltpu.sync_copy(dst_hbm.at[pl.ds(base, GR)], dstv)
            pltpu.async_copy(y_hbm.at[srcv.at[0]], rows, sem)

            def _pair(p, _):
                j0 = 2 * p
                pltpu.make_async_copy(y_hbm.at[srcv.at[j0]], rows, sem).wait()

                @pl.when(p > 0)
                def _():
                    pltpu.make_async_copy(
                        rows1, acc.at[dstv.at[j0 - 1]], ssem1).wait()

                pltpu.async_copy(y_hbm.at[srcv.at[j0 + 1]], rows1, sem1)
                pltpu.async_copy(rows, acc.at[dstv.at[j0]], ssem, add=True)
                if with_cnt:
                    pltpu.sync_copy(onesv.at[pl.ds(0, EC)],
                                    accc.at[dstv.at[j0]], add=True)
                pltpu.make_async_copy(
                    y_hbm.at[srcv.at[j0 + 1]], rows1, sem1).wait()
                pltpu.make_async_copy(rows, acc.at[dstv.at[j0]], ssem).wait()

                @pl.when(p < npair - 1)
                def _():
                    pltpu.async_copy(y_hbm.at[srcv.at[j0 + 2]], rows, sem)

                pltpu.async_copy(rows1, acc.at[dstv.at[j0 + 1]], ssem1,
                                 add=True)
                if with_cnt:
                    pltpu.sync_copy(onesv.at[pl.ds(0, EC)],
                                    accc.at[dstv.at[j0 + 1]], add=True)
                return 0

            lax.fori_loop(0, npair, _pair, 0)
            pltpu.make_async_copy(
                rows1, acc.at[dstv.at[GR - 1]], ssem1).wait()
            return 0

        with jax.named_scope("sc_edge_loop"):
            lax.fori_loop(0, RW // GR, _group, 0)

        with jax.named_scope("sc_wb"):
            plsc.subcore_barrier()
            pltpu.sync_copy(acc.at[pl.ds(sid * RPT, RPT)],
                            out_hbm.at[cid, pl.ds(sid * RPT, RPT)])
            if with_cnt:
                pltpu.sync_copy(accc.at[pl.ds(sid * RPT, RPT)],
                                cnt_hbm.at[cid, pl.ds(sid * RPT, RPT)])

    return pl.kernel(body, out_type=out_type, mesh=mesh, scratch_types=scratch)


_seg_sum_cnt_128 = _seg_sum_kernel(D_HID, True)
_seg_sum_128 = _seg_sum_kernel(D_HID, False)

_BR = 1000
_GRID = N // _BR


def _mm_body(x_ref, w_ref, o_ref):
    o_ref[...] = jnp.dot(x_ref[...], w_ref[...], preferred_element_type=jnp.float32)


def _tc_matmul(x, w):
    n, k = x.shape
    m = w.shape[1]
    return pl.pallas_call(
        _mm_body,
        grid=(_GRID,),
        in_specs=[pl.BlockSpec((_BR, k), lambda i: (i, 0)),
                  pl.BlockSpec((k, m), lambda i: (0, 0))],
        out_specs=pl.BlockSpec((_BR, m), lambda i: (i, 0)),
        out_shape=jax.ShapeDtypeStruct((n, m), jnp.float32),
    )(x, w)


def _make_comb_body(with_y):
    def _comb_body(s_ref, c_ref, h_ref, wr_ref, b_ref, g_ref, be_ref, rm_ref,
                   rv_ref, *rest):
        rc = 1.0 / jnp.maximum(c_ref[0] + c_ref[1], 1.0)
        agg = (s_ref[0] + s_ref[1]) * rc
        h = agg + jnp.dot(h_ref[...], wr_ref[...],
                          preferred_element_type=jnp.float32) + b_ref[...]
        scale = g_ref[...] * lax.rsqrt(rv_ref[...] + EPS)
        h = (h - rm_ref[...]) * scale + be_ref[...]
        h = jnp.maximum(h, 0.0)
        if with_y:
            wn_ref, h_out, y_out = rest
            h_out[...] = h
            y_out[...] = jnp.dot(h, wn_ref[...],
                                 preferred_element_type=jnp.float32)
        else:
            h_out, = rest
            h_out[...] = h
    return _comb_body


def _tc_combine(s, c, h, wr, b, g, be, rm, rv, wn=None):
    d = s.shape[2]
    vec = lambda: pl.BlockSpec((1, d), lambda i: (0, 0))
    in_specs = [pl.BlockSpec((NC, _BR, d), lambda i: (0, i, 0)),
                pl.BlockSpec((NC, _BR, 1), lambda i: (0, i, 0)),
                pl.BlockSpec((_BR, d), lambda i: (i, 0)),
                pl.BlockSpec((d, d), lambda i: (0, 0)),
                vec(), vec(), vec(), vec(), vec()]
    out_specs = [pl.BlockSpec((_BR, d), lambda i: (i, 0))]
    out_shape = [jax.ShapeDtypeStruct((N, d), jnp.float32)]
    args = (s, c, h, wr, b.reshape(1, d), g.reshape(1, d), be.reshape(1, d),
            rm.reshape(1, d), rv.reshape(1, d))
    if wn is not None:
        dn = wn.shape[1]
        in_specs.append(pl.BlockSpec((d, dn), lambda i: (0, 0)))
        out_specs = [pl.BlockSpec((_BR, d), lambda i: (i, 0)),
                     pl.BlockSpec((_BR, dn), lambda i: (i, 0))]
        out_shape = [jax.ShapeDtypeStruct((N, d), jnp.float32),
                     jax.ShapeDtypeStruct((N, dn), jnp.float32)]
        args = args + (wn,)
    return pl.pallas_call(
        _make_comb_body(wn is not None),
        grid=(_GRID,),
        in_specs=in_specs,
        out_specs=out_specs,
        out_shape=out_shape,
    )(*args)


def _fin_body(s_ref, c_ref, h_ref, wl_ref, wr_ref, b_ref, o_out):
    rc = 1.0 / jnp.maximum(c_ref[0] + c_ref[1], 1.0)
    agg = (s_ref[0] + s_ref[1]) * rc
    o = (jnp.dot(agg, wl_ref[...], preferred_element_type=jnp.float32)
         + jnp.dot(h_ref[...], wr_ref[...], preferred_element_type=jnp.float32)
         + b_ref[...])
    m = jnp.max(o, axis=1, keepdims=True)
    e = jnp.exp(o - m)
    o_out[...] = (o - m) - jnp.log(jnp.sum(e, axis=1, keepdims=True))


def _tc_final(s, c, h, wl, wr, b):
    d = h.shape[1]
    dn = wr.shape[1]
    return pl.pallas_call(
        _fin_body,
        grid=(_GRID,),
        in_specs=[pl.BlockSpec((NC, _BR, d), lambda i: (0, i, 0)),
                  pl.BlockSpec((NC, _BR, 1), lambda i: (0, i, 0)),
                  pl.BlockSpec((_BR, d), lambda i: (i, 0)),
                  pl.BlockSpec((d, dn), lambda i: (0, 0)),
                  pl.BlockSpec((d, dn), lambda i: (0, 0)),
                  pl.BlockSpec((1, dn), lambda i: (0, 0))],
        out_specs=pl.BlockSpec((_BR, dn), lambda i: (i, 0)),
        out_shape=jax.ShapeDtypeStruct((N, dn), jnp.float32),
    )(s, c, h, wl, wr, b.reshape(1, dn))


def kernel(x, edge_index, W0l, b0l, W0r, g0, be0, rm0, rv0,
           W1l, b1l, W1r, g1, be1, rm1, rv1, W2l, b2l, W2r):
    src = edge_index[0].reshape(EP // EC, EC)
    dst = edge_index[1].reshape(EP // EC, EC)

    y0 = _tc_matmul(x, W0l)
    s0, cnt = _seg_sum_cnt_128(y0, src, dst)
    cnt = cnt.reshape(NC, NP, 1)
    h1, y1 = _tc_combine(s0, cnt, x, W0r, b0l, g0, be0, rm0, rv0, W1l)
    s1, = _seg_sum_128(y1, src, dst)
    h2, = _tc_combine(s1, cnt, h1, W1r, b1l, g1, be1, rm1, rv1)
    s2, = _seg_sum_128(h2, src, dst)
    return _tc_final(s2, cnt, h2, W2l, W2r, b2l)
